# baseline (speedup 1.0000x reference)
import functools

import numpy as np
import jax
import jax.numpy as jnp
from jax.experimental import pallas as pl
from jax.experimental.pallas import tpu as pltpu

H = 512
W = 512
N2 = H * W
TOPK = 5
MIN_DIST = 10
SAL_THR = 0.1
KS = 5
SIGMA = 1.0
BW = 12
Q = 0.9
K_SEL = max(1, int((1.0 - Q) * N2))
ONE_BITS = 0x3F800001
NKC = 4


def _split3_np(a64):
    a1 = a64.astype(jnp.bfloat16)
    r1 = a64 - a1.astype(np.float64)
    a2 = r1.astype(jnp.bfloat16)
    r2 = r1 - a2.astype(np.float64)
    a3 = r2.astype(jnp.bfloat16)
    return np.stack([np.asarray(a1), np.asarray(a2), np.asarray(a3)])


def _build_consts():
    i = np.arange(H, dtype=np.int64)
    jk = np.outer(i, i) % H
    ang = (-2.0 * np.pi / H) * jk.astype(np.float64)
    C = np.cos(ang)
    S = np.sin(ang)

    ax = np.arange(KS, dtype=np.float32) - np.float32((KS - 1) / 2.0)
    g = np.exp(-(ax ** 2) / np.float32(2.0 * SIGMA ** 2)).astype(np.float32)
    g = (g / g.sum()).astype(np.float32)
    k2 = np.outer(g, g).astype(np.float32)
    k2b = k2.astype(jnp.bfloat16).astype(np.float32)

    y32 = np.arange(H, dtype=np.float32)
    wy = (np.float32(0.5) * (np.float32(1.0) - np.cos(
        (np.float32(2.0 * np.pi) * y32 / np.float32(H - 1)).astype(np.float32)
    ).astype(np.float32))).astype(np.float32)
    wm = np.outer(wy, wy).astype(np.float32)
    m1 = ((y32 >= BW) & (y32 < H - BW)).astype(np.float32)
    wm = wm * np.outer(m1, m1).astype(np.float32)
    return (_split3_np(C), _split3_np(S), wm, k2b)


_C3, _S3, _WM, _K2B = _build_consts()


def _split3(a):
    f32 = jnp.float32
    a1 = a.astype(jnp.bfloat16)
    r1 = a - a1.astype(f32)
    a2 = r1.astype(jnp.bfloat16)
    r2 = r1 - a2.astype(f32)
    a3 = r2.astype(jnp.bfloat16)
    return (a1, a2, a3)


def _mmb(a, b):
    return jax.lax.dot(a, b, preferred_element_type=jnp.float32)


def _mm_hp(a3, b3):
    a1, a2, a3 = a3[0], a3[1], a3[2]
    b1, b2, b3 = b3[0], b3[1], b3[2]
    acc = _mmb(a2, b2) + (_mmb(a1, b3) + _mmb(a3, b1))
    acc = acc + (_mmb(a1, b2) + _mmb(a2, b1))
    kc = H // NKC
    p = []
    for c in range(NKC):
        p.append(_mmb(a1[:, c * kc:(c + 1) * kc], b1[c * kc:(c + 1) * kc, :]))
    hi = (p[0] + p[1]) + (p[2] + p[3])
    return acc + hi


def _body(x_ref, c_ref, s_ref, wm_ref, sal_ref, pts_ref, pad_ref):
    f32 = jnp.float32
    xb = x_ref[0]
    gray = (xb[0] + xb[1] + xb[2]) * f32(1.0 / 3.0)

    C = c_ref[...]
    S = s_ref[...]

    g3 = _split3(gray)
    Tr = _mm_hp(C, g3)
    Ti = _mm_hp(S, g3)
    Tr3 = _split3(Tr)
    Ti3 = _split3(Ti)
    Fr = _mm_hp(Tr3, C) - _mm_hp(Ti3, S)
    Fi = _mm_hp(Tr3, S) + _mm_hp(Ti3, C)
    mag = jnp.sqrt(Fr * Fr + Fi * Fi) + f32(1e-8)
    Pr = Fr / mag
    Pi = Fi / mag

    Pr3 = _split3(Pr)
    Pi3 = _split3(Pi)
    Ur = _mm_hp(C, Pr3) + _mm_hp(S, Pi3)
    Ui = _mm_hp(C, Pi3) - _mm_hp(S, Pr3)
    Ur3 = _split3(Ur)
    Ui3 = _split3(Ui)
    Rr = _mm_hp(Ur3, C) + _mm_hp(Ui3, S)
    recon = Rr * f32(1.0 / N2)
    sal0 = recon * recon

    pad_ref[...] = jnp.zeros((H + 8, W + 128), dtype=f32)
    pad_ref[2:H + 2, 2:W + 2] = sal0.astype(jnp.bfloat16).astype(f32)
    sal2 = jnp.zeros((H, W), dtype=f32)
    for i in range(KS):
        for j in range(KS):
            sal2 = sal2 + f32(float(_K2B[i, j])) * pad_ref[i:i + H, j:j + W]
    sal2 = sal2 * wm_ref[...]

    mn = jnp.min(sal2)
    mx = jnp.max(sal2)
    saln = (sal2 - mn) / (mx - mn + f32(1e-8))
    sal_ref[0, 0] = saln

    bits = jax.lax.bitcast_convert_type(saln, jnp.int32)

    def bs_step(_, lohi):
        lo, hi = lohi
        mid = (lo + hi) // 2
        cnt = jnp.sum((bits >= mid).astype(jnp.int32))
        ge = cnt >= K_SEL
        return (jnp.where(ge, mid, lo), jnp.where(ge, hi, mid))

    lo, _ = jax.lax.fori_loop(0, 31, bs_step,
                              (jnp.int32(0), jnp.int32(ONE_BITS)))
    kth = jax.lax.bitcast_convert_type(lo, f32)
    thr = jnp.maximum(kth, f32(SAL_THR) * jnp.max(saln))

    ii = jax.lax.broadcasted_iota(jnp.int32, (H, W), 0)
    jj = jax.lax.broadcasted_iota(jnp.int32, (H, W), 1)
    flat_idx = ii * W + jj
    lane = jax.lax.broadcasted_iota(jnp.int32, (1, 128), 1)
    riota = jax.lax.broadcasted_iota(jnp.int32, (8, 128), 0)

    work = saln
    acc = jnp.zeros((8, 128), dtype=f32)
    for t in range(TOPK):
        m = jnp.max(work)
        idx = jnp.min(jnp.where(work == m, flat_idx, jnp.int32(N2)))
        py = idx // W
        px = idx - py * W
        ok = m > thr
        pxf = jnp.where(ok, px.astype(f32), f32(-1.0))
        pyf = jnp.where(ok, py.astype(f32), f32(-1.0))
        labf = jnp.where(ok, f32(1.0), f32(-1.0))
        row = jnp.where(lane == 0, pxf,
                        jnp.where(lane == 1, pyf,
                                  jnp.where(lane == 2, labf, f32(0.0))))
        acc = jnp.where(riota == t, row, acc)
        dy = ii - py
        dx = jj - px
        supp = (dy * dy + dx * dx) <= (MIN_DIST * MIN_DIST)
        work = jnp.where(supp, f32(-jnp.inf), work)
    pts_ref[0] = acc


@jax.jit
def kernel(x):
    B = x.shape[0]
    sal, pts = pl.pallas_call(
        _body,
        grid=(B,),
        in_specs=[
            pl.BlockSpec((1, 3, H, W), lambda i: (i, 0, 0, 0)),
            pl.BlockSpec((3, H, W), lambda i: (0, 0, 0)),
            pl.BlockSpec((3, H, W), lambda i: (0, 0, 0)),
            pl.BlockSpec((H, W), lambda i: (0, 0)),
        ],
        out_specs=[
            pl.BlockSpec((1, 1, H, W), lambda i: (i, 0, 0, 0)),
            pl.BlockSpec((1, 8, 128), lambda i: (i, 0, 0)),
        ],
        out_shape=[
            jax.ShapeDtypeStruct((B, 1, H, W), jnp.float32),
            jax.ShapeDtypeStruct((B, 8, 128), jnp.float32),
        ],
        scratch_shapes=[pltpu.VMEM((H + 8, W + 128), jnp.float32)],
    )(x, _C3, _S3, _WM)
    coords = pts[:, :TOPK, :2]
    labels = pts[:, :TOPK, 2].astype(jnp.int32)
    return coords, labels, sal


# speedup vs baseline: 1.2432x; 1.2432x over previous
import functools

import numpy as np
import jax
import jax.numpy as jnp
from jax.experimental import pallas as pl
from jax.experimental.pallas import tpu as pltpu

H = 512
W = 512
N2 = H * W
TOPK = 5
MIN_DIST = 10
SAL_THR = 0.1
KS = 5
SIGMA = 1.0
BW = 12
Q = 0.9
K_SEL = max(1, int((1.0 - Q) * N2))
ONE_BITS = 0x3F800001
NKC = 4
HH = 264


def _split3_np(a64):
    a1 = a64.astype(jnp.bfloat16)
    r1 = a64 - a1.astype(np.float64)
    a2 = r1.astype(jnp.bfloat16)
    r2 = r1 - a2.astype(np.float64)
    a3 = r2.astype(jnp.bfloat16)
    return np.stack([np.asarray(a1), np.asarray(a2), np.asarray(a3)])


def _build_consts():
    i = np.arange(H, dtype=np.int64)
    jk = np.outer(i, i) % H
    ang = (-2.0 * np.pi / H) * jk.astype(np.float64)
    C = np.cos(ang)
    S = np.sin(ang)

    ax = np.arange(KS, dtype=np.float32) - np.float32((KS - 1) / 2.0)
    g = np.exp(-(ax ** 2) / np.float32(2.0 * SIGMA ** 2)).astype(np.float32)
    g = (g / g.sum()).astype(np.float32)
    k2 = np.outer(g, g).astype(np.float32)
    k2b = k2.astype(jnp.bfloat16).astype(np.float32)

    y32 = np.arange(H, dtype=np.float32)
    wy = (np.float32(0.5) * (np.float32(1.0) - np.cos(
        (np.float32(2.0 * np.pi) * y32 / np.float32(H - 1)).astype(np.float32)
    ).astype(np.float32))).astype(np.float32)
    wm = np.outer(wy, wy).astype(np.float32)
    m1 = ((y32 >= BW) & (y32 < H - BW)).astype(np.float32)
    wm = wm * np.outer(m1, m1).astype(np.float32)
    return (_split3_np(C), _split3_np(S), wm, k2b)


_C3, _S3, _WM, _K2B = _build_consts()


def _split3(a):
    f32 = jnp.float32
    a1 = a.astype(jnp.bfloat16)
    r1 = a - a1.astype(f32)
    a2 = r1.astype(jnp.bfloat16)
    r2 = r1 - a2.astype(f32)
    a3 = r2.astype(jnp.bfloat16)
    return (a1, a2, a3)


def _mmb(a, b):
    return jax.lax.dot(a, b, preferred_element_type=jnp.float32)


def _mm_hp(a3, b3):
    a1, a2, a3 = a3[0], a3[1], a3[2]
    b1, b2, b3 = b3[0], b3[1], b3[2]
    acc = _mmb(a2, b2) + (_mmb(a1, b3) + _mmb(a3, b1))
    acc = acc + (_mmb(a1, b2) + _mmb(a2, b1))
    k = a1.shape[1]
    nkc = max(1, k // 128)
    kc = k // nkc
    p = []
    for c in range(nkc):
        p.append(_mmb(a1[:, c * kc:(c + 1) * kc], b1[c * kc:(c + 1) * kc, :]))
    while len(p) > 1:
        p = [p[i] + p[i + 1] for i in range(0, len(p) - 1, 2)] + (
            [p[-1]] if len(p) % 2 else [])
    return acc + p[0]


def _body(x_ref, c_ref, s_ref, wm_ref, sal_ref, pts_ref, pad_ref):
    f32 = jnp.float32
    xb = x_ref[0]
    gray = (xb[0] + xb[1] + xb[2]) * f32(1.0 / 3.0)

    C = c_ref[...]
    S = s_ref[...]
    Ch = C[:, :HH, :]
    Sh = S[:, :HH, :]
    Ck = C[:, :, :HH]
    Sk = S[:, :, :HH]

    g3 = _split3(gray)
    Tr = _mm_hp(Ch, g3)
    Ti = _mm_hp(Sh, g3)
    Tr3 = _split3(Tr)
    Ti3 = _split3(Ti)
    Fr = _mm_hp(Tr3, C) - _mm_hp(Ti3, S)
    Fi = _mm_hp(Tr3, S) + _mm_hp(Ti3, C)
    mag = jnp.sqrt(Fr * Fr + Fi * Fi) + f32(1e-8)
    Pr = Fr / mag
    Pi = Fi / mag

    Pr3 = _split3(Pr)
    Pi3 = _split3(Pi)
    Vr = _mm_hp(Pr3, C) + _mm_hp(Pi3, S)
    Vi = _mm_hp(Pi3, C) - _mm_hp(Pr3, S)

    rw = jax.lax.broadcasted_iota(jnp.int32, (HH, 1), 0)
    wgt = jnp.where((rw == 0) | (rw == H // 2), f32(1.0),
                    jnp.where(rw < H // 2, f32(2.0), f32(0.0)))
    Vr3 = _split3(Vr * wgt)
    Vi3 = _split3(Vi * wgt)
    Rr = _mm_hp(Ck, Vr3) + _mm_hp(Sk, Vi3)
    recon = Rr * f32(1.0 / N2)
    sal0 = recon * recon

    pad_ref[...] = jnp.zeros((H + 8, W + 128), dtype=f32)
    pad_ref[2:H + 2, 2:W + 2] = sal0.astype(jnp.bfloat16).astype(f32)
    sal2 = jnp.zeros((H, W), dtype=f32)
    for i in range(KS):
        for j in range(KS):
            sal2 = sal2 + f32(float(_K2B[i, j])) * pad_ref[i:i + H, j:j + W]
    sal2 = sal2 * wm_ref[...]

    mn = jnp.min(sal2)
    mx = jnp.max(sal2)
    saln = (sal2 - mn) / (mx - mn + f32(1e-8))
    sal_ref[0, 0] = saln

    bits = jax.lax.bitcast_convert_type(saln, jnp.int32)

    def bs_step(_, lohi):
        lo, hi = lohi
        mid = (lo + hi) // 2
        cnt = jnp.sum((bits >= mid).astype(jnp.int32))
        ge = cnt >= K_SEL
        return (jnp.where(ge, mid, lo), jnp.where(ge, hi, mid))

    lo, _ = jax.lax.fori_loop(0, 31, bs_step,
                              (jnp.int32(0), jnp.int32(ONE_BITS)))
    kth = jax.lax.bitcast_convert_type(lo, f32)
    thr = jnp.maximum(kth, f32(SAL_THR) * jnp.max(saln))

    ii = jax.lax.broadcasted_iota(jnp.int32, (H, W), 0)
    jj = jax.lax.broadcasted_iota(jnp.int32, (H, W), 1)
    flat_idx = ii * W + jj
    lane = jax.lax.broadcasted_iota(jnp.int32, (1, 128), 1)
    riota = jax.lax.broadcasted_iota(jnp.int32, (8, 128), 0)

    work = saln
    acc = jnp.zeros((8, 128), dtype=f32)
    for t in range(TOPK):
        m = jnp.max(work)
        idx = jnp.min(jnp.where(work == m, flat_idx, jnp.int32(N2)))
        py = idx // W
        px = idx - py * W
        ok = m > thr
        pxf = jnp.where(ok, px.astype(f32), f32(-1.0))
        pyf = jnp.where(ok, py.astype(f32), f32(-1.0))
        labf = jnp.where(ok, f32(1.0), f32(-1.0))
        row = jnp.where(lane == 0, pxf,
                        jnp.where(lane == 1, pyf,
                                  jnp.where(lane == 2, labf, f32(0.0))))
        acc = jnp.where(riota == t, row, acc)
        dy = ii - py
        dx = jj - px
        supp = (dy * dy + dx * dx) <= (MIN_DIST * MIN_DIST)
        work = jnp.where(supp, f32(-jnp.inf), work)
    pts_ref[0] = acc


@jax.jit
def kernel(x):
    B = x.shape[0]
    sal, pts = pl.pallas_call(
        _body,
        grid=(B,),
        in_specs=[
            pl.BlockSpec((1, 3, H, W), lambda i: (i, 0, 0, 0)),
            pl.BlockSpec((3, H, W), lambda i: (0, 0, 0)),
            pl.BlockSpec((3, H, W), lambda i: (0, 0, 0)),
            pl.BlockSpec((H, W), lambda i: (0, 0)),
        ],
        out_specs=[
            pl.BlockSpec((1, 1, H, W), lambda i: (i, 0, 0, 0)),
            pl.BlockSpec((1, 8, 128), lambda i: (i, 0, 0)),
        ],
        out_shape=[
            jax.ShapeDtypeStruct((B, 1, H, W), jnp.float32),
            jax.ShapeDtypeStruct((B, 8, 128), jnp.float32),
        ],
        scratch_shapes=[pltpu.VMEM((H + 8, W + 128), jnp.float32)],
    )(x, _C3, _S3, _WM)
    coords = pts[:, :TOPK, :2]
    labels = pts[:, :TOPK, 2].astype(jnp.int32)
    return coords, labels, sal


# speedup vs baseline: 1.4608x; 1.1750x over previous
import functools

import numpy as np
import jax
import jax.numpy as jnp
from jax.experimental import pallas as pl
from jax.experimental.pallas import tpu as pltpu

H = 512
W = 512
N2 = H * W
TOPK = 5
MIN_DIST = 10
SAL_THR = 0.1
KS = 5
SIGMA = 1.0
BW = 12
Q = 0.9
K_SEL = max(1, int((1.0 - Q) * N2))
ONE_BITS = 0x3F800001
NKC = 4
HH = 264


def _split3_np(a64):
    a1 = a64.astype(jnp.bfloat16)
    r1 = a64 - a1.astype(np.float64)
    a2 = r1.astype(jnp.bfloat16)
    r2 = r1 - a2.astype(np.float64)
    a3 = r2.astype(jnp.bfloat16)
    return np.stack([np.asarray(a1), np.asarray(a2), np.asarray(a3)])


def _build_consts():
    i = np.arange(H, dtype=np.int64)
    jk = np.outer(i, i) % H
    ang = (-2.0 * np.pi / H) * jk.astype(np.float64)
    C = np.cos(ang)
    S = np.sin(ang)

    ax = np.arange(KS, dtype=np.float32) - np.float32((KS - 1) / 2.0)
    g = np.exp(-(ax ** 2) / np.float32(2.0 * SIGMA ** 2)).astype(np.float32)
    g = (g / g.sum()).astype(np.float32)
    k2 = np.outer(g, g).astype(np.float32)
    k2b = np.asarray(k2.astype(jnp.bfloat16), dtype=np.float32)

    cc = np.arange(H)[:, None]
    ll = np.arange(W)[None, :]
    jj = cc - ll + 2
    Bb = np.zeros((KS, H, W), dtype=np.float32)
    for i in range(KS):
        valid = (jj >= 0) & (jj < KS)
        Bb[i][valid] = k2b[i, jj[valid]]
    Bb = Bb.astype(jnp.bfloat16)

    y32 = np.arange(H, dtype=np.float32)
    wy = (np.float32(0.5) * (np.float32(1.0) - np.cos(
        (np.float32(2.0 * np.pi) * y32 / np.float32(H - 1)).astype(np.float32)
    ).astype(np.float32))).astype(np.float32)
    wm = np.outer(wy, wy).astype(np.float32)
    m1 = ((y32 >= BW) & (y32 < H - BW)).astype(np.float32)
    wm = wm * np.outer(m1, m1).astype(np.float32)
    return (_split3_np(C), _split3_np(S), wm, Bb)


_C3, _S3, _WM, _BB = _build_consts()


def _split3(a):
    f32 = jnp.float32
    a1 = a.astype(jnp.bfloat16)
    r1 = a - a1.astype(f32)
    a2 = r1.astype(jnp.bfloat16)
    r2 = r1 - a2.astype(f32)
    a3 = r2.astype(jnp.bfloat16)
    return (a1, a2, a3)


def _mmb(a, b):
    return jax.lax.dot(a, b, preferred_element_type=jnp.float32)


def _mm_hp(a3, b3):
    a1, a2, a3 = a3[0], a3[1], a3[2]
    b1, b2, b3 = b3[0], b3[1], b3[2]
    acc = _mmb(a2, b2) + (_mmb(a1, b3) + _mmb(a3, b1))
    acc = acc + (_mmb(a1, b2) + _mmb(a2, b1))
    k = a1.shape[1]
    nkc = max(1, k // 128)
    kc = k // nkc
    p = []
    for c in range(nkc):
        p.append(_mmb(a1[:, c * kc:(c + 1) * kc], b1[c * kc:(c + 1) * kc, :]))
    while len(p) > 1:
        p = [p[i] + p[i + 1] for i in range(0, len(p) - 1, 2)] + (
            [p[-1]] if len(p) % 2 else [])
    return acc + p[0]


def _body(x_ref, c_ref, s_ref, wm_ref, b_ref, sal_ref, pts_ref, pad_ref):
    f32 = jnp.float32
    xb = x_ref[0]
    gray = (xb[0] + xb[1] + xb[2]) * f32(1.0 / 3.0)

    C = c_ref[...]
    S = s_ref[...]
    Ch = C[:, :HH, :]
    Sh = S[:, :HH, :]
    Ck = C[:, :, :HH]
    Sk = S[:, :, :HH]

    g3 = _split3(gray)
    Tr = _mm_hp(Ch, g3)
    Ti = _mm_hp(Sh, g3)
    Tr3 = _split3(Tr)
    Ti3 = _split3(Ti)
    Fr = _mm_hp(Tr3, C) - _mm_hp(Ti3, S)
    Fi = _mm_hp(Tr3, S) + _mm_hp(Ti3, C)
    mag = jnp.sqrt(Fr * Fr + Fi * Fi) + f32(1e-8)
    Pr = Fr / mag
    Pi = Fi / mag

    Pr3 = _split3(Pr)
    Pi3 = _split3(Pi)
    Vr = _mm_hp(Pr3, C) + _mm_hp(Pi3, S)
    Vi = _mm_hp(Pi3, C) - _mm_hp(Pr3, S)

    rw = jax.lax.broadcasted_iota(jnp.int32, (HH, 1), 0)
    wgt = jnp.where((rw == 0) | (rw == H // 2), f32(1.0),
                    jnp.where(rw < H // 2, f32(2.0), f32(0.0)))
    Vr3 = _split3(Vr * wgt)
    Vi3 = _split3(Vi * wgt)
    Rr = _mm_hp(Ck, Vr3) + _mm_hp(Sk, Vi3)
    recon = Rr * f32(1.0 / N2)
    sal0 = recon * recon

    pad_ref[...] = jnp.zeros((H + 8, W), dtype=f32)
    pad_ref[2:H + 2, :] = sal0.astype(jnp.bfloat16).astype(f32)
    sal2 = jnp.zeros((H, W), dtype=f32)
    for i in range(KS):
        zi = pad_ref[i:i + H, :].astype(jnp.bfloat16)
        sal2 = sal2 + _mmb(zi, b_ref[i])
    sal2 = sal2 * wm_ref[...]

    mn = jnp.min(sal2)
    mx = jnp.max(sal2)
    saln = (sal2 - mn) / (mx - mn + f32(1e-8))
    sal_ref[0, 0] = saln

    bits = jax.lax.bitcast_convert_type(saln, jnp.int32)

    def bs_step(_, lohi):
        lo, hi = lohi
        mid = (lo + hi) // 2
        cnt = jnp.sum((bits >= mid).astype(jnp.int32))
        ge = cnt >= K_SEL
        return (jnp.where(ge, mid, lo), jnp.where(ge, hi, mid))

    lo, _ = jax.lax.fori_loop(0, 31, bs_step,
                              (jnp.int32(0), jnp.int32(ONE_BITS)))
    kth = jax.lax.bitcast_convert_type(lo, f32)
    thr = jnp.maximum(kth, f32(SAL_THR) * jnp.max(saln))

    ii = jax.lax.broadcasted_iota(jnp.int32, (H, W), 0)
    jj = jax.lax.broadcasted_iota(jnp.int32, (H, W), 1)
    flat_idx = ii * W + jj
    lane = jax.lax.broadcasted_iota(jnp.int32, (1, 128), 1)
    riota = jax.lax.broadcasted_iota(jnp.int32, (8, 128), 0)

    work = saln
    acc = jnp.zeros((8, 128), dtype=f32)
    for t in range(TOPK):
        m = jnp.max(work)
        idx = jnp.min(jnp.where(work == m, flat_idx, jnp.int32(N2)))
        py = idx // W
        px = idx - py * W
        ok = m > thr
        pxf = jnp.where(ok, px.astype(f32), f32(-1.0))
        pyf = jnp.where(ok, py.astype(f32), f32(-1.0))
        labf = jnp.where(ok, f32(1.0), f32(-1.0))
        row = jnp.where(lane == 0, pxf,
                        jnp.where(lane == 1, pyf,
                                  jnp.where(lane == 2, labf, f32(0.0))))
        acc = jnp.where(riota == t, row, acc)
        dy = ii - py
        dx = jj - px
        supp = (dy * dy + dx * dx) <= (MIN_DIST * MIN_DIST)
        work = jnp.where(supp, f32(-jnp.inf), work)
    pts_ref[0] = acc


@jax.jit
def kernel(x):
    B = x.shape[0]
    sal, pts = pl.pallas_call(
        _body,
        grid=(B,),
        in_specs=[
            pl.BlockSpec((1, 3, H, W), lambda i: (i, 0, 0, 0)),
            pl.BlockSpec((3, H, W), lambda i: (0, 0, 0)),
            pl.BlockSpec((3, H, W), lambda i: (0, 0, 0)),
            pl.BlockSpec((H, W), lambda i: (0, 0)),
            pl.BlockSpec((KS, H, W), lambda i: (0, 0, 0)),
        ],
        out_specs=[
            pl.BlockSpec((1, 1, H, W), lambda i: (i, 0, 0, 0)),
            pl.BlockSpec((1, 8, 128), lambda i: (i, 0, 0)),
        ],
        out_shape=[
            jax.ShapeDtypeStruct((B, 1, H, W), jnp.float32),
            jax.ShapeDtypeStruct((B, 8, 128), jnp.float32),
        ],
        scratch_shapes=[pltpu.VMEM((H + 8, W), jnp.float32)],
    )(x, _C3, _S3, _WM, _BB)
    coords = pts[:, :TOPK, :2]
    labels = pts[:, :TOPK, 2].astype(jnp.int32)
    return coords, labels, sal


# speedup vs baseline: 1.5352x; 1.0509x over previous
import functools

import numpy as np
import jax
import jax.numpy as jnp
from jax.experimental import pallas as pl
from jax.experimental.pallas import tpu as pltpu

H = 512
W = 512
N2 = H * W
TOPK = 5
MIN_DIST = 10
SAL_THR = 0.1
KS = 5
SIGMA = 1.0
BW = 12
Q = 0.9
K_SEL = max(1, int((1.0 - Q) * N2))
ONE_BITS = 0x3F800001
NKC = 4
HH = 264
NB = 2


def _split3_np(a64):
    a1 = a64.astype(jnp.bfloat16)
    r1 = a64 - a1.astype(np.float64)
    a2 = r1.astype(jnp.bfloat16)
    r2 = r1 - a2.astype(np.float64)
    a3 = r2.astype(jnp.bfloat16)
    return np.stack([np.asarray(a1), np.asarray(a2), np.asarray(a3)])


def _build_consts():
    i = np.arange(H, dtype=np.int64)
    jk = np.outer(i, i) % H
    ang = (-2.0 * np.pi / H) * jk.astype(np.float64)
    C = np.cos(ang)
    S = np.sin(ang)

    ax = np.arange(KS, dtype=np.float32) - np.float32((KS - 1) / 2.0)
    g = np.exp(-(ax ** 2) / np.float32(2.0 * SIGMA ** 2)).astype(np.float32)
    g = (g / g.sum()).astype(np.float32)
    k2 = np.outer(g, g).astype(np.float32)
    k2b = np.asarray(k2.astype(jnp.bfloat16), dtype=np.float32)

    cc = np.arange(H)[:, None]
    ll = np.arange(W)[None, :]
    jj = cc - ll + 2
    Bb = np.zeros((KS, H, W), dtype=np.float32)
    for i in range(KS):
        valid = (jj >= 0) & (jj < KS)
        Bb[i][valid] = k2b[i, jj[valid]]
    Bb = Bb.astype(jnp.bfloat16)

    y32 = np.arange(H, dtype=np.float32)
    wy = (np.float32(0.5) * (np.float32(1.0) - np.cos(
        (np.float32(2.0 * np.pi) * y32 / np.float32(H - 1)).astype(np.float32)
    ).astype(np.float32))).astype(np.float32)
    wm = np.outer(wy, wy).astype(np.float32)
    m1 = ((y32 >= BW) & (y32 < H - BW)).astype(np.float32)
    wm = wm * np.outer(m1, m1).astype(np.float32)
    return (_split3_np(C), _split3_np(S), wm, Bb)


_C3, _S3, _WM, _BB = _build_consts()


def _split3(a):
    f32 = jnp.float32
    a1 = a.astype(jnp.bfloat16)
    r1 = a - a1.astype(f32)
    a2 = r1.astype(jnp.bfloat16)
    r2 = r1 - a2.astype(f32)
    a3 = r2.astype(jnp.bfloat16)
    return (a1, a2, a3)


def _mmb(a, b):
    return jax.lax.dot(a, b, preferred_element_type=jnp.float32)


def _mm_hp(a3, b3):
    a1, a2, a3 = a3[0], a3[1], a3[2]
    b1, b2, b3 = b3[0], b3[1], b3[2]
    acc = _mmb(a2, b2) + (_mmb(a1, b3) + _mmb(a3, b1))
    acc = acc + (_mmb(a1, b2) + _mmb(a2, b1))
    k = a1.shape[1]
    nkc = max(1, k // 128)
    kc = k // nkc
    p = []
    for c in range(nkc):
        p.append(_mmb(a1[:, c * kc:(c + 1) * kc], b1[c * kc:(c + 1) * kc, :]))
    while len(p) > 1:
        p = [p[i] + p[i + 1] for i in range(0, len(p) - 1, 2)] + (
            [p[-1]] if len(p) % 2 else [])
    return acc + p[0]


def _body(x_ref, c_ref, s_ref, wm_ref, b_ref, sal_ref, pts_ref, pad_ref):
    for b in range(NB):
        _image(b, x_ref, c_ref, s_ref, wm_ref, b_ref, sal_ref, pts_ref,
               pad_ref)


def _image(b, x_ref, c_ref, s_ref, wm_ref, b_ref, sal_ref, pts_ref, pad_ref):
    f32 = jnp.float32
    xb = x_ref[b]
    gray = (xb[0] + xb[1] + xb[2]) * f32(1.0 / 3.0)

    C = c_ref[...]
    S = s_ref[...]
    Ch = C[:, :HH, :]
    Sh = S[:, :HH, :]
    Ck = C[:, :, :HH]
    Sk = S[:, :, :HH]

    g3 = _split3(gray)
    Tr = _mm_hp(Ch, g3)
    Ti = _mm_hp(Sh, g3)
    Tr3 = _split3(Tr)
    Ti3 = _split3(Ti)
    Fr = _mm_hp(Tr3, C) - _mm_hp(Ti3, S)
    Fi = _mm_hp(Tr3, S) + _mm_hp(Ti3, C)
    mag = jnp.sqrt(Fr * Fr + Fi * Fi) + f32(1e-8)
    Pr = Fr / mag
    Pi = Fi / mag

    Pr3 = _split3(Pr)
    Pi3 = _split3(Pi)
    Vr = _mm_hp(Pr3, C) + _mm_hp(Pi3, S)
    Vi = _mm_hp(Pi3, C) - _mm_hp(Pr3, S)

    rw = jax.lax.broadcasted_iota(jnp.int32, (HH, 1), 0)
    wgt = jnp.where((rw == 0) | (rw == H // 2), f32(1.0),
                    jnp.where(rw < H // 2, f32(2.0), f32(0.0)))
    Vr3 = _split3(Vr * wgt)
    Vi3 = _split3(Vi * wgt)
    Rr = _mm_hp(Ck, Vr3) + _mm_hp(Sk, Vi3)
    recon = Rr * f32(1.0 / N2)
    sal0 = recon * recon

    pad_ref[b] = jnp.zeros((H + 8, W), dtype=f32)
    pad_ref[b, 2:H + 2, :] = sal0.astype(jnp.bfloat16).astype(f32)
    sal2 = jnp.zeros((H, W), dtype=f32)
    for i in range(KS):
        zi = pad_ref[b, i:i + H, :].astype(jnp.bfloat16)
        sal2 = sal2 + _mmb(zi, b_ref[i])
    sal2 = sal2 * wm_ref[...]

    mn = jnp.min(sal2)
    mx = jnp.max(sal2)
    saln = (sal2 - mn) / (mx - mn + f32(1e-8))
    sal_ref[b, 0] = saln

    bits = jax.lax.bitcast_convert_type(saln, jnp.int32)

    def bs_step(_, lohi):
        lo, hi = lohi
        mid = (lo + hi) // 2
        cnt = jnp.sum((bits >= mid).astype(jnp.int32))
        ge = cnt >= K_SEL
        return (jnp.where(ge, mid, lo), jnp.where(ge, hi, mid))

    lo, _ = jax.lax.fori_loop(0, 31, bs_step,
                              (jnp.int32(0), jnp.int32(ONE_BITS)))
    kth = jax.lax.bitcast_convert_type(lo, f32)
    thr = jnp.maximum(kth, f32(SAL_THR) * jnp.max(saln))

    ii = jax.lax.broadcasted_iota(jnp.int32, (H, W), 0)
    jj = jax.lax.broadcasted_iota(jnp.int32, (H, W), 1)
    flat_idx = ii * W + jj
    lane = jax.lax.broadcasted_iota(jnp.int32, (1, 128), 1)
    riota = jax.lax.broadcasted_iota(jnp.int32, (8, 128), 0)

    work = saln
    acc = jnp.zeros((8, 128), dtype=f32)
    for t in range(TOPK):
        m = jnp.max(work)
        idx = jnp.min(jnp.where(work == m, flat_idx, jnp.int32(N2)))
        py = idx // W
        px = idx - py * W
        ok = m > thr
        pxf = jnp.where(ok, px.astype(f32), f32(-1.0))
        pyf = jnp.where(ok, py.astype(f32), f32(-1.0))
        labf = jnp.where(ok, f32(1.0), f32(-1.0))
        row = jnp.where(lane == 0, pxf,
                        jnp.where(lane == 1, pyf,
                                  jnp.where(lane == 2, labf, f32(0.0))))
        acc = jnp.where(riota == t, row, acc)
        dy = ii - py
        dx = jj - px
        supp = (dy * dy + dx * dx) <= (MIN_DIST * MIN_DIST)
        work = jnp.where(supp, f32(-jnp.inf), work)
    pts_ref[b] = acc


@jax.jit
def kernel(x):
    B = x.shape[0]
    sal, pts = pl.pallas_call(
        _body,
        grid=(B // NB,),
        in_specs=[
            pl.BlockSpec((NB, 3, H, W), lambda i: (i, 0, 0, 0)),
            pl.BlockSpec((3, H, W), lambda i: (0, 0, 0)),
            pl.BlockSpec((3, H, W), lambda i: (0, 0, 0)),
            pl.BlockSpec((H, W), lambda i: (0, 0)),
            pl.BlockSpec((KS, H, W), lambda i: (0, 0, 0)),
        ],
        out_specs=[
            pl.BlockSpec((NB, 1, H, W), lambda i: (i, 0, 0, 0)),
            pl.BlockSpec((NB, 8, 128), lambda i: (i, 0, 0)),
        ],
        out_shape=[
            jax.ShapeDtypeStruct((B, 1, H, W), jnp.float32),
            jax.ShapeDtypeStruct((B, 8, 128), jnp.float32),
        ],
        scratch_shapes=[pltpu.VMEM((NB, H + 8, W), jnp.float32)],
    )(x, _C3, _S3, _WM, _BB)
    coords = pts[:, :TOPK, :2]
    labels = pts[:, :TOPK, 2].astype(jnp.int32)
    return coords, labels, sal
